# baseline (speedup 1.0000x reference)
import functools

import jax
import jax.numpy as jnp
import numpy as np
from jax import lax
from jax.experimental import pallas as pl
from jax.experimental.pallas import tpu as pltpu
from jax.experimental.pallas import tpu_sc as plsc

N = 10000
E = 320000
IN_SIZE = 128
OUT_SIZE = 64
HEADS = 8
HID = 128
HD = HEADS * OUT_SIZE
NMP = 2
NG = 4
GW = 2 * OUT_SIZE
NTILE = 16
EP = 327680
ROWS_E = EP // 128
TROWS = ROWS_E // NTILE
WROWS = 8
NWIN = TROWS // WROWS
CHUNK = 128
CH = 624
TAIL = N - NTILE * CH
ROWBLK = 1000

_j = np.arange(HD)
_gj = _j // 128
_jj = _j % 128
_FEAT_STD = (((_jj >> 1) & 7) * OUT_SIZE + 16 * _gj
             + ((_jj & 1) + 2 * ((_jj >> 4) & 7)))
_PERM2 = np.concatenate([_FEAT_STD, HD + _FEAT_STD])
_p = np.arange(HD)
_pp = _p % 128
_i32 = _pp % 32
_jacc = (_p // 128) * 128 + 32 * (_pp // 32) + np.where(
    _i32 < 16, 2 * _i32, 2 * (_i32 - 16) + 1)
_ACC_STD = ((((_jacc % 128) >> 1) & 7) * OUT_SIZE + 16 * (_jacc // 128)
            + (((_jacc % 128) & 1) + 2 * (((_jacc % 128) >> 4) & 7)))
_UNPERM = np.zeros((HD, HD), np.float32)
_UNPERM[np.arange(HD), _ACC_STD] = 1.0
_REP = np.zeros((HEADS, HD), np.float32)
_REP[(_p % 16) & 7, _p] = 1.0


def _dense_body(x_ref, w_ref, a_ref, feat_ref, elx_ref, erx_ref):
    fb = jnp.dot(x_ref[...], w_ref[...], preferred_element_type=jnp.float32)
    eb = jnp.dot(fb, a_ref[...], preferred_element_type=jnp.float32)
    for q in range(NMP * NG):
        feat_ref[q] = fb[:, q * GW:(q + 1) * GW].astype(jnp.bfloat16)
    for m_ in range(NMP):
        el = eb[:, 16 * m_:16 * m_ + 8]
        er = eb[:, 16 * m_ + 8:16 * m_ + 16]
        elx_ref[m_] = jnp.concatenate([el, el], axis=1)
        erx_ref[m_] = jnp.concatenate([er, er], axis=1)


def _sc_body(ei_ref, elx_ref, erx_ref, feat_ref, zro_ref, zro8_ref,
             acc_o, ssum_o, ex_o, acc_s, ssum_s, gsems, ssems):
    c = lax.axis_index("c")
    s = lax.axis_index("s")
    c02 = jnp.full((16,), 0.2, jnp.float32)
    zf = jnp.zeros((16,), jnp.float32)
    rbase = s * TROWS

    def p0_scope(src2d, dst2d, sbufs, dbufs, xb, exflatD):
        pltpu.sync_copy(zro8_ref, ssum_s.at[pl.ds(s * CH, CH), :])

        @pl.when(s == 0)
        def _():
            pltpu.sync_copy(zro8_ref.at[pl.ds(0, TAIL), :],
                            ssum_s.at[pl.ds(NTILE * CH, TAIL), :])

        plsc.subcore_barrier()

        def p0_issue(r, par):
            pltpu.async_copy(elx_ref.at[c].at[src2d.at[r]], sbufs[par],
                             gsems[par])
            pltpu.async_copy(erx_ref.at[c].at[dst2d.at[r]], dbufs[par],
                             gsems[par])

        def p0_window(w, carry):
            wrow = rbase + w * WROWS
            pltpu.sync_copy(ei_ref.at[2 * c, pl.ds(wrow, WROWS), :], src2d)
            pltpu.sync_copy(ei_ref.at[2 * c + 1, pl.ds(wrow, WROWS), :],
                            dst2d)
            p0_issue(0, 0)
            for r in range(WROWS):
                par = r & 1
                if r + 1 < WROWS:
                    p0_issue(r + 1, 1 - par)
                pltpu.make_async_copy(elx_ref.at[c].at[src2d.at[r]],
                                      sbufs[par], gsems[par]).wait()
                pltpu.make_async_copy(erx_ref.at[c].at[dst2d.at[r]],
                                      dbufs[par], gsems[par]).wait()
                padf = jnp.full(
                    (16,),
                    jnp.where(wrow + r < E // CHUNK, 1.0, 0.0).astype(
                        jnp.float32))
                sb, db = sbufs[par], dbufs[par]

                def one(b):
                    v = sb[b, :] + db[b, :]
                    v = jnp.maximum(v, zf) + c02 * jnp.minimum(v, zf)
                    v = jnp.exp(v) * padf
                    xb[b, :] = v
                    return v

                def pair(j, c2):
                    va = one(2 * j)
                    vb = one(2 * j + 1)
                    exflatD[r, pl.ds(32 * j, 32)] = plsc.pack(
                        va, vb, format=plsc.PackFormat.INTERLEAVED)
                    return c2

                lax.fori_loop(0, CHUNK // 2, pair, 0, unroll=2)
                pltpu.sync_copy(xb, ssum_s.at[dst2d.at[r]], add=True)
            pltpu.sync_copy(exflatD, ex_o.at[c, pl.ds(wrow, WROWS), :])
            return carry

        lax.fori_loop(0, NWIN, p0_window, 0)
        plsc.subcore_barrier()
        pltpu.sync_copy(ssum_s.at[pl.ds(s * CH, CH), :],
                        ssum_o.at[c, pl.ds(s * CH, CH), :])

        @pl.when(s == 0)
        def _():
            pltpu.sync_copy(ssum_s.at[pl.ds(NTILE * CH, TAIL), :],
                            ssum_o.at[c, pl.ds(NTILE * CH, TAIL), :])

        plsc.subcore_barrier()

    pl.run_scoped(
        p0_scope,
        pltpu.VMEM((WROWS, CHUNK), jnp.int32),
        pltpu.VMEM((WROWS, CHUNK), jnp.int32),
        [pltpu.VMEM((CHUNK, 16), jnp.float32) for _ in range(2)],
        [pltpu.VMEM((CHUNK, 16), jnp.float32) for _ in range(2)],
        pltpu.VMEM((CHUNK, 16), jnp.float32),
        pltpu.VMEM((WROWS, 16 * CHUNK), jnp.bfloat16),
    )

    def gp_scope(src2d, dst2d, exflatD, rowbufs, outb):
        def swait():
            pltpu.make_async_copy(outb, acc_s.at[dst2d.at[0]],
                                  ssems[0]).wait()

        for g in range(NG):
            ftab = feat_ref.at[c * NG + g]
            pltpu.sync_copy(zro_ref, acc_s.at[pl.ds(s * CH, CH), :])

            @pl.when(s == 0)
            def _():
                pltpu.sync_copy(zro_ref.at[pl.ds(0, TAIL), :],
                                acc_s.at[pl.ds(NTILE * CH, TAIL), :])

            plsc.subcore_barrier()

            def gp_window(w, carry):
                wrow = rbase + w * WROWS
                @pl.when(w > 0)
                def _():
                    swait()

                pltpu.sync_copy(ei_ref.at[2 * c, pl.ds(wrow, WROWS), :],
                                src2d)
                pltpu.sync_copy(ei_ref.at[2 * c + 1, pl.ds(wrow, WROWS), :],
                                dst2d)
                pltpu.async_copy(ftab.at[src2d.at[0]], rowbufs[0], gsems[0])
                for r in range(WROWS):
                    par = r & 1
                    if r & 3 == 0:
                        pltpu.sync_copy(
                            ex_o.at[c, pl.ds(wrow + r, 4), :], exflatD)
                    if r + 1 < WROWS:
                        pltpu.async_copy(ftab.at[src2d.at[r + 1]],
                                         rowbufs[1 - par], gsems[1 - par])
                    rb = rowbufs[par]
                    pltpu.make_async_copy(ftab.at[src2d.at[r]], rb,
                                          gsems[par]).wait()
                    if r >= 1:
                        swait()

                    def mpair(j, c2):
                        exw = exflatD[r & 3, pl.ds(32 * j, 32)]
                        e0, e1 = plsc.unpack(
                            exw, format=plsc.PackFormat.INTERLEAVED)
                        for bb, ee in ((2 * j, e0), (2 * j + 1, e1)):
                            for k in range(4):
                                wv = rb[bb, pl.ds(32 * k, 32)]
                                lo, hi = plsc.unpack(
                                    wv, format=plsc.PackFormat.INTERLEAVED)
                                outb[bb, pl.ds(32 * k, 16)] = lo * ee
                                outb[bb, pl.ds(32 * k + 16, 16)] = hi * ee
                        return c2

                    lax.fori_loop(0, CHUNK // 2, mpair, 0, unroll=2)
                    pltpu.async_copy(outb, acc_s.at[dst2d.at[r]], ssems[0],
                                     add=True)
                return carry

            lax.fori_loop(0, NWIN, gp_window, 0)
            swait()
            plsc.subcore_barrier()
            pltpu.sync_copy(acc_s.at[pl.ds(s * CH, CH), :],
                            acc_o.at[c * NG + g, pl.ds(s * CH, CH), :])

            @pl.when(s == 0)
            def _():
                pltpu.sync_copy(
                    acc_s.at[pl.ds(NTILE * CH, TAIL), :],
                    acc_o.at[c * NG + g, pl.ds(NTILE * CH, TAIL), :])

            plsc.subcore_barrier()

    pl.run_scoped(
        gp_scope,
        pltpu.VMEM((WROWS, CHUNK), jnp.int32),
        pltpu.VMEM((WROWS, CHUNK), jnp.int32),
        pltpu.VMEM((4, 16 * CHUNK), jnp.bfloat16),
        [pltpu.VMEM((CHUNK, GW), jnp.bfloat16) for _ in range(2)],
        pltpu.VMEM((CHUNK, GW), jnp.float32),
    )


_sc_call = functools.partial(
    pl.kernel,
    out_type=(
        jax.ShapeDtypeStruct((NMP * NG, N, GW), jnp.float32),
        jax.ShapeDtypeStruct((NMP, N, 16), jnp.float32),
        jax.ShapeDtypeStruct((NMP, ROWS_E, 16 * CHUNK), jnp.bfloat16),
    ),
    mesh=plsc.VectorSubcoreMesh(core_axis_name="c", subcore_axis_name="s"),
    compiler_params=pltpu.CompilerParams(use_tc_tiling_on_sc=False,
                                         needs_layout_passes=False),
    scratch_types=[
        pltpu.VMEM_SHARED((N, GW), jnp.float32),
        pltpu.VMEM_SHARED((N, 16), jnp.float32),
        [pltpu.SemaphoreType.DMA for _ in range(2)],
        [pltpu.SemaphoreType.DMA for _ in range(2)],
    ],
)(_sc_body)


def _post_body(acc_ref, ssum_ref, sw1_ref, sb1_ref, sw2_ref, rep_ref,
               z_ref, w_ref):
    i = pl.program_id(0)
    qs = []
    for m_ in range(NMP):
        accm = jnp.concatenate([acc_ref[m_ * NG + g] for g in range(NG)],
                               axis=1)
        accm = jnp.maximum(accm, 0.0) + 0.01 * jnp.minimum(accm, 0.0)
        recip = 1.0 / (ssum_ref[m_][:, 0:HEADS] + 1e-30)
        rrep = jnp.dot(recip, rep_ref[...],
                       preferred_element_type=jnp.float32)
        zm = accm * rrep
        z_ref[m_] = zm
        t = jnp.tanh(jnp.dot(zm, sw1_ref[...],
                             preferred_element_type=jnp.float32)
                     + sb1_ref[...])
        qs.append(jnp.sum(t * sw2_ref[...], axis=0, keepdims=True))
    q = jnp.concatenate(qs, axis=0)

    @pl.when(i == 0)
    def _():
        w_ref[...] = q

    @pl.when(i != 0)
    def _():
        w_ref[...] = w_ref[...] + q


def _comb_body(z_ref, b_ref, p_ref, o_ref):
    zc = b_ref[0, 0] * z_ref[0] + b_ref[0, 1] * z_ref[1]
    o_ref[...] = jnp.dot(zc, p_ref[...], preferred_element_type=jnp.float32)


def kernel(x, edge_index0, edge_index1, W0, al0, ar0, W1, al1, ar1,
           sW1, sb1, sW2):
    f32 = jnp.float32
    Wcat = jnp.concatenate([W0, W1], axis=1)
    eye8 = jnp.eye(HEADS, dtype=f32)
    maskc = jnp.kron(eye8, jnp.ones((OUT_SIZE, 1), f32))
    A = jnp.zeros((2 * HD, 4 * HEADS), f32)
    A = A.at[:HD, 0:8].set(maskc * al0.reshape(-1)[:, None])
    A = A.at[:HD, 8:16].set(maskc * ar0.reshape(-1)[:, None])
    A = A.at[HD:, 16:24].set(maskc * al1.reshape(-1)[:, None])
    A = A.at[HD:, 24:32].set(maskc * ar1.reshape(-1)[:, None])
    perm2 = jnp.asarray(_PERM2)
    Wcat = Wcat[:, perm2]
    A = A[perm2, :]

    nblk = N // ROWBLK
    feat_t, elx_t, erx_t = pl.pallas_call(
        _dense_body,
        grid=(nblk,),
        in_specs=[
            pl.BlockSpec((ROWBLK, IN_SIZE), lambda i: (i, 0)),
            pl.BlockSpec((IN_SIZE, NMP * HD), lambda i: (0, 0)),
            pl.BlockSpec((NMP * HD, 4 * HEADS), lambda i: (0, 0)),
        ],
        out_specs=[
            pl.BlockSpec((NMP * NG, ROWBLK, GW), lambda i: (0, i, 0)),
            pl.BlockSpec((NMP, ROWBLK, 16), lambda i: (0, i, 0)),
            pl.BlockSpec((NMP, ROWBLK, 16), lambda i: (0, i, 0)),
        ],
        out_shape=[
            jax.ShapeDtypeStruct((NMP * NG, N, GW), jnp.bfloat16),
            jax.ShapeDtypeStruct((NMP, N, 16), f32),
            jax.ShapeDtypeStruct((NMP, N, 16), f32),
        ],
    )(x, Wcat, A)

    pad = jnp.zeros((EP - E,), jnp.int32)
    ei = jnp.stack([
        jnp.concatenate([edge_index0[0], pad]).reshape(ROWS_E, CHUNK),
        jnp.concatenate([edge_index0[1], pad]).reshape(ROWS_E, CHUNK),
        jnp.concatenate([edge_index1[0], pad]).reshape(ROWS_E, CHUNK),
        jnp.concatenate([edge_index1[1], pad]).reshape(ROWS_E, CHUNK),
    ], axis=0)
    zros = jnp.zeros((CH, GW), f32)
    zros8 = jnp.zeros((CH, 16), f32)
    acc, ssum, _exo = _sc_call(ei, elx_t, erx_t, feat_t, zros, zros8)

    rep = jnp.asarray(_REP)
    sW1p = sW1[jnp.asarray(_ACC_STD), :]
    z, wacc = pl.pallas_call(
        _post_body,
        grid=(nblk,),
        in_specs=[
            pl.BlockSpec((NMP * NG, ROWBLK, GW), lambda i: (0, i, 0)),
            pl.BlockSpec((NMP, ROWBLK, 16), lambda i: (0, i, 0)),
            pl.BlockSpec((HD, HID), lambda i: (0, 0)),
            pl.BlockSpec((1, HID), lambda i: (0, 0)),
            pl.BlockSpec((1, HID), lambda i: (0, 0)),
            pl.BlockSpec((HEADS, HD), lambda i: (0, 0)),
        ],
        out_specs=[
            pl.BlockSpec((NMP, ROWBLK, HD), lambda i: (0, i, 0)),
            pl.BlockSpec((NMP, HID), lambda i: (0, 0)),
        ],
        out_shape=[
            jax.ShapeDtypeStruct((NMP, N, HD), f32),
            jax.ShapeDtypeStruct((NMP, HID), f32),
        ],
    )(acc, ssum, sW1p, sb1.reshape(1, HID), sW2.reshape(1, HID), rep)

    w2 = wacc.sum(axis=1) / N
    beta = jax.nn.softmax(w2)
    out = pl.pallas_call(
        _comb_body,
        grid=(nblk,),
        in_specs=[
            pl.BlockSpec((NMP, ROWBLK, HD), lambda i: (0, i, 0)),
            pl.BlockSpec((1, NMP), lambda i: (0, 0)),
            pl.BlockSpec((HD, HD), lambda i: (0, 0)),
        ],
        out_specs=pl.BlockSpec((ROWBLK, HD), lambda i: (i, 0)),
        out_shape=jax.ShapeDtypeStruct((N, HD), f32),
    )(z, beta.reshape(1, NMP), jnp.asarray(_UNPERM))
    return out
